# Initial kernel scaffold; baseline (speedup 1.0000x reference)
import jax
import jax.numpy as jnp
from jax.experimental import pallas as pl


def kernel(x_user, x_movie, edge_label_index):
    raise NotImplementedError("write your pallas kernel here")


# speedup vs baseline: 1.4603x; 1.4603x over previous
import functools
import math

import jax
import jax.numpy as jnp
from jax import lax
from jax.experimental import pallas as pl
from jax.experimental.pallas import tpu as pltpu
from jax.experimental.pallas import tpu_sc as plsc

D = 64
L = 16
NC = 2
NS = 16
NW = NC * NS
C = 256


@functools.partial(jax.jit, static_argnums=(3,))
def _run(x_user, x_movie, eli, T):
    P = NW * T * C
    E = eli.shape[1]
    idx = jnp.zeros((2, P), jnp.int32).at[:, :E].set(eli.astype(jnp.int32))
    mesh = plsc.VectorSubcoreMesh(core_axis_name="c", subcore_axis_name="s")

    @functools.partial(
        pl.kernel,
        mesh=mesh,
        compiler_params=pltpu.CompilerParams(
            needs_layout_passes=False, use_tc_tiling_on_sc=False
        ),
        out_type=jax.ShapeDtypeStruct((P,), jnp.float32),
        scratch_types=[
            pltpu.VMEM((C,), jnp.int32),
            pltpu.VMEM((C,), jnp.int32),
            pltpu.VMEM((C, D), jnp.float32),
            pltpu.VMEM((C, D), jnp.float32),
            pltpu.VMEM((C,), jnp.float32),
            pltpu.SemaphoreType.DMA,
        ],
    )
    def k(xu_hbm, xm_hbm, iu_hbm, im_hbm, out_hbm, iu_v, im_v, ru_v, rm_v, o_v, sem):
        wid = lax.axis_index("s") * NC + lax.axis_index("c")

        def chunk_body(t, carry):
            base = (wid * T + t) * C
            pltpu.sync_copy(iu_hbm.at[pl.ds(base, C)], iu_v)
            pltpu.sync_copy(im_hbm.at[pl.ds(base, C)], im_v)
            cp_u = pltpu.async_copy(xu_hbm.at[iu_v], ru_v, sem)
            cp_m = pltpu.async_copy(xm_hbm.at[im_v], rm_v, sem)
            cp_u.wait()
            cp_m.wait()

            lanes = lax.iota(jnp.int32, L)

            def group_body(g, carry2):
                rows = g * L + lanes
                acc0 = jnp.zeros((L,), jnp.float32)
                acc1 = jnp.zeros((L,), jnp.float32)
                for d in range(0, D, 2):
                    c0 = jnp.full((L,), d, jnp.int32)
                    c1 = jnp.full((L,), d + 1, jnp.int32)
                    acc0 = acc0 + plsc.load_gather(ru_v, [rows, c0]) * plsc.load_gather(rm_v, [rows, c0])
                    acc1 = acc1 + plsc.load_gather(ru_v, [rows, c1]) * plsc.load_gather(rm_v, [rows, c1])
                o_v[pl.ds(g * L, L)] = acc0 + acc1
                return carry2

            lax.fori_loop(0, C // L, group_body, 0)
            pltpu.sync_copy(o_v, out_hbm.at[pl.ds(base, C)])
            return carry

        lax.fori_loop(0, T, chunk_body, 0)

    out = k(x_user, x_movie, idx[0], idx[1])
    return out[:E]


def kernel(x_user, x_movie, edge_label_index):
    E = edge_label_index.shape[1]
    T = math.ceil(E / (NW * C))
    return _run(x_user, x_movie, edge_label_index, T)


# speedup vs baseline: 3.1925x; 2.1862x over previous
import functools
import math

import jax
import jax.numpy as jnp
from jax import lax
from jax.experimental import pallas as pl
from jax.experimental.pallas import tpu as pltpu
from jax.experimental.pallas import tpu_sc as plsc

D = 64
L = 16
NC = 2
NS = 16
NW = NC * NS
C = 256


@functools.partial(jax.jit, static_argnums=(3,))
def _run(x_user, x_movie, eli, T):
    P = NW * T * C
    E = eli.shape[1]
    idx = jnp.zeros((2, P), jnp.int32).at[:, :E].set(eli.astype(jnp.int32))
    mesh = plsc.VectorSubcoreMesh(core_axis_name="c", subcore_axis_name="s")

    @functools.partial(
        pl.kernel,
        mesh=mesh,
        compiler_params=pltpu.CompilerParams(
            needs_layout_passes=False, use_tc_tiling_on_sc=False
        ),
        out_type=jax.ShapeDtypeStruct((P,), jnp.float32),
        scratch_types=[
            pltpu.VMEM((C,), jnp.int32),
            pltpu.VMEM((C,), jnp.int32),
            pltpu.VMEM((C, D), jnp.float32),
            pltpu.VMEM((C, D), jnp.float32),
            pltpu.VMEM((C,), jnp.float32),
            pltpu.SemaphoreType.DMA,
        ],
    )
    def k(xu_hbm, xm_hbm, iu_hbm, im_hbm, out_hbm, iu_v, im_v, ru_v, rm_v, o_v, sem):
        wid = lax.axis_index("s") * NC + lax.axis_index("c")

        def chunk_body(t, carry):
            base = (wid * T + t) * C
            pltpu.sync_copy(iu_hbm.at[pl.ds(base, C)], iu_v)
            pltpu.sync_copy(im_hbm.at[pl.ds(base, C)], im_v)
            cp_u = pltpu.async_copy(xu_hbm.at[iu_v], ru_v, sem)
            cp_m = pltpu.async_copy(xm_hbm.at[im_v], rm_v, sem)
            cp_u.wait()
            cp_m.wait()

            lanes = lax.iota(jnp.int32, L)
            iE = (lanes % (L // 2)) * 2
            iO = iE + 1
            lo_mask = lanes < (L // 2)

            def _perm(a, idx):
                return jnp.take_along_axis(a, idx, axis=0)

            def _hadd(a, b):
                ta = _perm(a, iE) + _perm(a, iO)
                tb = _perm(b, iE) + _perm(b, iO)
                return jnp.where(lo_mask, ta, tb)

            def group_body(g, carry2):
                eb = g * L
                p = []
                for j in range(L):
                    e = eb + j
                    v = ru_v[e, pl.ds(0, L)] * rm_v[e, pl.ds(0, L)]
                    for q in range(1, D // L):
                        v = v + ru_v[e, pl.ds(q * L, L)] * rm_v[e, pl.ds(q * L, L)]
                    p.append(v)
                while len(p) > 1:
                    p = [_hadd(p[i], p[i + 1]) for i in range(0, len(p), 2)]
                o_v[pl.ds(eb, L)] = p[0]
                return carry2

            lax.fori_loop(0, C // L, group_body, 0)
            pltpu.sync_copy(o_v, out_hbm.at[pl.ds(base, C)])
            return carry

        lax.fori_loop(0, T, chunk_body, 0)

    out = k(x_user, x_movie, idx[0], idx[1])
    return out[:E]


def kernel(x_user, x_movie, edge_label_index):
    E = edge_label_index.shape[1]
    T = math.ceil(E / (NW * C))
    return _run(x_user, x_movie, edge_label_index, T)


# speedup vs baseline: 4.8084x; 1.5061x over previous
import functools
import math

import jax
import jax.numpy as jnp
from jax import lax
from jax.experimental import pallas as pl
from jax.experimental.pallas import tpu as pltpu
from jax.experimental.pallas import tpu_sc as plsc

D = 64
L = 16
NC = 2
NS = 16
NW = NC * NS
C = 256


@functools.partial(jax.jit, static_argnums=(3,))
def _run(x_user, x_movie, eli, T):
    P = NW * T * C
    E = eli.shape[1]
    idx = jnp.zeros((2, P), jnp.int32).at[:, :E].set(eli.astype(jnp.int32))
    mesh = plsc.VectorSubcoreMesh(core_axis_name="c", subcore_axis_name="s")

    @functools.partial(
        pl.kernel,
        mesh=mesh,
        compiler_params=pltpu.CompilerParams(
            needs_layout_passes=False, use_tc_tiling_on_sc=False
        ),
        out_type=jax.ShapeDtypeStruct((P,), jnp.float32),
        scratch_types=[
            pltpu.VMEM((T * C,), jnp.int32),
            pltpu.VMEM((T * C,), jnp.int32),
            pltpu.VMEM((C, D), jnp.float32),
            pltpu.VMEM((C, D), jnp.float32),
            pltpu.VMEM((C, D), jnp.float32),
            pltpu.VMEM((C, D), jnp.float32),
            pltpu.VMEM((C,), jnp.float32),
            pltpu.VMEM((C,), jnp.float32),
            pltpu.SemaphoreType.DMA,
            pltpu.SemaphoreType.DMA,
            pltpu.SemaphoreType.DMA,
            pltpu.SemaphoreType.DMA,
        ],
    )
    def k(xu, xm, iu, im, out, iu_all, im_all,
          ru0, ru1, rm0, rm1, o0, o1, sg0, sg1, so0, so1):
        wid = lax.axis_index("s") * NC + lax.axis_index("c")
        wbase = wid * (T * C)
        pltpu.sync_copy(iu.at[pl.ds(wbase, T * C)], iu_all)
        pltpu.sync_copy(im.at[pl.ds(wbase, T * C)], im_all)
        bufs = ((ru0, rm0, o0, sg0, so0), (ru1, rm1, o1, sg1, so1))

        def gathers(t, b):
            ruv, rmv, _, sg, _ = bufs[b]
            cu = pltpu.make_async_copy(xu.at[iu_all.at[pl.ds(t * C, C)]], ruv, sg)
            cm = pltpu.make_async_copy(xm.at[im_all.at[pl.ds(t * C, C)]], rmv, sg)
            return cu, cm

        for b in range(2):
            cu, cm = gathers(b, b)
            cu.start()
            cm.start()

        lanes = lax.iota(jnp.int32, L)
        iE = (lanes % (L // 2)) * 2
        iO = iE + 1
        lo_mask = lanes < (L // 2)

        def _perm(a, idx_):
            return jnp.take_along_axis(a, idx_, axis=0)

        def _hadd(a, b):
            ta = _perm(a, iE) + _perm(a, iO)
            tb = _perm(b, iE) + _perm(b, iO)
            return jnp.where(lo_mask, ta, tb)

        def outer(i, carry):
            t0 = i * 2
            for b in range(2):
                ruv, rmv, ov, sg, so = bufs[b]
                t = t0 + b
                base = wbase + t * C
                cu, cm = gathers(t, b)
                cu.wait()
                cm.wait()

                @pl.when(t >= 2)
                def _():
                    pltpu.make_async_copy(
                        ov, out.at[pl.ds(base - 2 * C, C)], so
                    ).wait()

                def group_body(g, carry2):
                    eb = g * L
                    p = []
                    for j in range(L):
                        e = eb + j
                        v = ruv[e, pl.ds(0, L)] * rmv[e, pl.ds(0, L)]
                        for q in range(1, D // L):
                            v = v + ruv[e, pl.ds(q * L, L)] * rmv[e, pl.ds(q * L, L)]
                        p.append(v)
                    while len(p) > 1:
                        p = [_hadd(p[i2], p[i2 + 1]) for i2 in range(0, len(p), 2)]
                    ov[pl.ds(eb, L)] = p[0]
                    return carry2

                lax.fori_loop(0, C // L, group_body, 0)
                pltpu.make_async_copy(ov, out.at[pl.ds(base, C)], so).start()

                @pl.when(t + 2 < T)
                def _():
                    cu2, cm2 = gathers(t + 2, b)
                    cu2.start()
                    cm2.start()

            return carry

        lax.fori_loop(0, T // 2, outer, 0)

        for b in range(2):
            _, _, ov, _, so = bufs[b]
            t = T - 2 + b
            pltpu.make_async_copy(ov, out.at[pl.ds(wbase + t * C, C)], so).wait()

    out = k(x_user, x_movie, idx[0], idx[1])
    return out[:E]


def kernel(x_user, x_movie, edge_label_index):
    E = edge_label_index.shape[1]
    T = math.ceil(E / (NW * C))
    T += T % 2
    return _run(x_user, x_movie, edge_label_index, T)


# speedup vs baseline: 6.3584x; 1.3224x over previous
import functools
import math

import jax
import jax.numpy as jnp
from jax import lax
from jax.experimental import pallas as pl
from jax.experimental.pallas import tpu as pltpu
from jax.experimental.pallas import tpu_sc as plsc

D = 64
L = 16
NC = 2
NS = 16
NW = NC * NS
C = 256


@functools.partial(jax.jit, static_argnums=(3,))
def _run(x_user, x_movie, eli, T):
    P = NW * T * C
    E = eli.shape[1]
    idx = jnp.zeros((2, P), jnp.int32).at[:, :E].set(eli.astype(jnp.int32))
    x_user = x_user.astype(jnp.bfloat16)
    x_movie = x_movie.astype(jnp.bfloat16)
    mesh = plsc.VectorSubcoreMesh(core_axis_name="c", subcore_axis_name="s")

    @functools.partial(
        pl.kernel,
        mesh=mesh,
        compiler_params=pltpu.CompilerParams(
            needs_layout_passes=False, use_tc_tiling_on_sc=False
        ),
        out_type=jax.ShapeDtypeStruct((P,), jnp.float32),
        scratch_types=[
            pltpu.VMEM((T * C,), jnp.int32),
            pltpu.VMEM((T * C,), jnp.int32),
            pltpu.VMEM((C, D), jnp.bfloat16),
            pltpu.VMEM((C, D), jnp.bfloat16),
            pltpu.VMEM((C, D), jnp.bfloat16),
            pltpu.VMEM((C, D), jnp.bfloat16),
            pltpu.VMEM((C,), jnp.float32),
            pltpu.VMEM((C,), jnp.float32),
            pltpu.SemaphoreType.DMA,
            pltpu.SemaphoreType.DMA,
            pltpu.SemaphoreType.DMA,
            pltpu.SemaphoreType.DMA,
        ],
    )
    def k(xu, xm, iu, im, out, iu_all, im_all,
          ru0, ru1, rm0, rm1, o0, o1, sg0, sg1, so0, so1):
        wid = lax.axis_index("s") * NC + lax.axis_index("c")
        wbase = wid * (T * C)
        pltpu.sync_copy(iu.at[pl.ds(wbase, T * C)], iu_all)
        pltpu.sync_copy(im.at[pl.ds(wbase, T * C)], im_all)
        bufs = ((ru0, rm0, o0, sg0, so0), (ru1, rm1, o1, sg1, so1))

        def gathers(t, b):
            ruv, rmv, _, sg, _ = bufs[b]
            cu = pltpu.make_async_copy(xu.at[iu_all.at[pl.ds(t * C, C)]], ruv, sg)
            cm = pltpu.make_async_copy(xm.at[im_all.at[pl.ds(t * C, C)]], rmv, sg)
            return cu, cm

        for b in range(2):
            cu, cm = gathers(b, b)
            cu.start()
            cm.start()

        lanes = lax.iota(jnp.int32, L)
        iE = (lanes % (L // 2)) * 2
        iO = iE + 1
        lo_mask = lanes < (L // 2)

        def _perm(a, idx_):
            return jnp.take_along_axis(a, idx_, axis=0)

        def _hadd(a, b):
            ta = _perm(a, iE) + _perm(a, iO)
            tb = _perm(b, iE) + _perm(b, iO)
            return jnp.where(lo_mask, ta, tb)

        def outer(i, carry):
            t0 = i * 2
            for b in range(2):
                ruv, rmv, ov, sg, so = bufs[b]
                t = t0 + b
                base = wbase + t * C
                cu, cm = gathers(t, b)
                cu.wait()
                cm.wait()

                @pl.when(t >= 2)
                def _():
                    pltpu.make_async_copy(
                        ov, out.at[pl.ds(base - 2 * C, C)], so
                    ).wait()

                def group_body(g, carry2):
                    eb = g * L
                    p = []
                    for j in range(L):
                        e = eb + j
                        v = None
                        for q in range(D // (2 * L)):
                            u0, u1 = plsc.unpack(
                                ruv[e, pl.ds(q * 2 * L, 2 * L)],
                                format=plsc.PackFormat.INTERLEAVED,
                            )
                            m0, m1 = plsc.unpack(
                                rmv[e, pl.ds(q * 2 * L, 2 * L)],
                                format=plsc.PackFormat.INTERLEAVED,
                            )
                            w = u0 * m0 + u1 * m1
                            v = w if v is None else v + w
                        p.append(v)
                    while len(p) > 1:
                        p = [_hadd(p[i2], p[i2 + 1]) for i2 in range(0, len(p), 2)]
                    ov[pl.ds(eb, L)] = p[0]
                    return carry2

                lax.fori_loop(0, C // L, group_body, 0)
                pltpu.make_async_copy(ov, out.at[pl.ds(base, C)], so).start()

                @pl.when(t + 2 < T)
                def _():
                    cu2, cm2 = gathers(t + 2, b)
                    cu2.start()
                    cm2.start()

            return carry

        lax.fori_loop(0, T // 2, outer, 0)

        for b in range(2):
            _, _, ov, _, so = bufs[b]
            t = T - 2 + b
            pltpu.make_async_copy(ov, out.at[pl.ds(wbase + t * C, C)], so).wait()

    out = k(x_user, x_movie, idx[0], idx[1])
    return out[:E]


def kernel(x_user, x_movie, edge_label_index):
    E = edge_label_index.shape[1]
    T = math.ceil(E / (NW * C))
    T += T % 2
    return _run(x_user, x_movie, edge_label_index, T)


# speedup vs baseline: 8.4545x; 1.3297x over previous
import functools

import jax
import jax.numpy as jnp
from jax import lax
from jax.experimental import pallas as pl
from jax.experimental.pallas import tpu as pltpu
from jax.experimental.pallas import tpu_sc as plsc

D = 64
L = 16
NC = 2
NS = 16
NW = NC * NS
C = 256


@jax.jit
def _run(x_user, x_movie, eli):
    E = eli.shape[1]
    eli = eli.astype(jnp.int32)
    x_user = x_user.astype(jnp.bfloat16)
    x_movie = x_movie.astype(jnp.bfloat16)
    Tf = E // (NW * C)
    T = Tf + 1
    T += T % 2
    assert NW * C <= E
    mesh = plsc.VectorSubcoreMesh(core_axis_name="c", subcore_axis_name="s")

    @functools.partial(
        pl.kernel,
        mesh=mesh,
        compiler_params=pltpu.CompilerParams(
            needs_layout_passes=False, use_tc_tiling_on_sc=False
        ),
        out_type=jax.ShapeDtypeStruct((E,), jnp.float32),
        scratch_types=[
            pltpu.VMEM((T * C,), jnp.int32),
            pltpu.VMEM((T * C,), jnp.int32),
            pltpu.VMEM((C, D), jnp.bfloat16),
            pltpu.VMEM((C, D), jnp.bfloat16),
            pltpu.VMEM((C, D), jnp.bfloat16),
            pltpu.VMEM((C, D), jnp.bfloat16),
            pltpu.VMEM((C,), jnp.float32),
            pltpu.VMEM((C,), jnp.float32),
            pltpu.SemaphoreType.DMA,
            pltpu.SemaphoreType.DMA,
            pltpu.SemaphoreType.DMA,
            pltpu.SemaphoreType.DMA,
        ],
    )
    def k(xu, xm, ei, out, iu_all, im_all,
          ru0, ru1, rm0, rm1, o0, o1, sg0, sg1, so0, so1):
        wid = lax.axis_index("s") * NC + lax.axis_index("c")
        n_tail = T - Tf

        def hbase(t):
            return jnp.where(
                t < Tf,
                (wid * Tf + t) * C,
                E - ((T - t) * NW - wid) * C,
            )

        pltpu.sync_copy(ei.at[0, pl.ds(wid * Tf * C, Tf * C)],
                        iu_all.at[pl.ds(0, Tf * C)])
        pltpu.sync_copy(ei.at[1, pl.ds(wid * Tf * C, Tf * C)],
                        im_all.at[pl.ds(0, Tf * C)])
        for j in range(n_tail):
            tb = E - ((n_tail - j) * NW - wid) * C
            pltpu.sync_copy(ei.at[0, pl.ds(tb, C)],
                            iu_all.at[pl.ds((Tf + j) * C, C)])
            pltpu.sync_copy(ei.at[1, pl.ds(tb, C)],
                            im_all.at[pl.ds((Tf + j) * C, C)])

        bufs = ((ru0, rm0, o0, sg0, so0), (ru1, rm1, o1, sg1, so1))

        def gathers(t, b):
            ruv, rmv, _, sg, _ = bufs[b]
            cu = pltpu.make_async_copy(xu.at[iu_all.at[pl.ds(t * C, C)]], ruv, sg)
            cm = pltpu.make_async_copy(xm.at[im_all.at[pl.ds(t * C, C)]], rmv, sg)
            return cu, cm

        for b in range(2):
            cu, cm = gathers(b, b)
            cu.start()
            cm.start()

        lanes = lax.iota(jnp.int32, L)
        iE = (lanes % (L // 2)) * 2
        iO = iE + 1
        lo_mask = lanes < (L // 2)

        def _perm(a, idx_):
            return jnp.take_along_axis(a, idx_, axis=0)

        def _hadd(a, b):
            ta = _perm(a, iE) + _perm(a, iO)
            tb = _perm(b, iE) + _perm(b, iO)
            return jnp.where(lo_mask, ta, tb)

        def outer(i, carry):
            t0 = i * 2
            for b in range(2):
                ruv, rmv, ov, sg, so = bufs[b]
                t = t0 + b
                cu, cm = gathers(t, b)
                cu.wait()
                cm.wait()

                @pl.when(t >= 2)
                def _():
                    pltpu.make_async_copy(
                        ov, out.at[pl.ds(hbase(t - 2), C)], so
                    ).wait()

                def group_body(g, carry2):
                    eb = g * L
                    p = []
                    for j in range(L):
                        e = eb + j
                        v = None
                        for q in range(D // (2 * L)):
                            u0, u1 = plsc.unpack(
                                ruv[e, pl.ds(q * 2 * L, 2 * L)],
                                format=plsc.PackFormat.INTERLEAVED,
                            )
                            m0, m1 = plsc.unpack(
                                rmv[e, pl.ds(q * 2 * L, 2 * L)],
                                format=plsc.PackFormat.INTERLEAVED,
                            )
                            w = u0 * m0 + u1 * m1
                            v = w if v is None else v + w
                        p.append(v)
                    while len(p) > 1:
                        p = [_hadd(p[i2], p[i2 + 1]) for i2 in range(0, len(p), 2)]
                    ov[pl.ds(eb, L)] = p[0]
                    return carry2

                lax.fori_loop(0, C // L, group_body, 0)
                pltpu.make_async_copy(ov, out.at[pl.ds(hbase(t), C)], so).start()

                @pl.when(t + 2 < T)
                def _():
                    cu2, cm2 = gathers(t + 2, b)
                    cu2.start()
                    cm2.start()

            return carry

        lax.fori_loop(0, T // 2, outer, 0)

        for b in range(2):
            _, _, ov, _, so = bufs[b]
            t = T - 2 + b
            pltpu.make_async_copy(ov, out.at[pl.ds(hbase(t), C)], so).wait()

    return k(x_user, x_movie, eli)


def kernel(x_user, x_movie, edge_label_index):
    return _run(x_user, x_movie, edge_label_index)


# speedup vs baseline: 8.9127x; 1.0542x over previous
import functools

import jax
import jax.numpy as jnp
from jax import lax
from jax.experimental import pallas as pl
from jax.experimental.pallas import tpu as pltpu
from jax.experimental.pallas import tpu_sc as plsc

D = 64
L = 16
NC = 2
NS = 16
NW = NC * NS
C = 256


@jax.jit
def _run(x_user, x_movie, eli):
    E = eli.shape[1]
    eli = eli.astype(jnp.int32)
    Tf = E // (NW * C)
    T = Tf + 1
    T += T % 2
    assert NW * C <= E
    mesh = plsc.VectorSubcoreMesh(core_axis_name="c", subcore_axis_name="s")

    @functools.partial(
        pl.kernel,
        mesh=mesh,
        compiler_params=pltpu.CompilerParams(
            needs_layout_passes=False, use_tc_tiling_on_sc=False
        ),
        out_type=jax.ShapeDtypeStruct((E,), jnp.float32),
        scratch_types=[
            pltpu.VMEM((T * C,), jnp.int32),
            pltpu.VMEM((T * C,), jnp.int32),
            pltpu.VMEM((C, D), jnp.float32),
            pltpu.VMEM((C, D), jnp.float32),
            pltpu.VMEM((C, D), jnp.float32),
            pltpu.VMEM((C, D), jnp.float32),
            pltpu.VMEM((C,), jnp.float32),
            pltpu.VMEM((C,), jnp.float32),
            pltpu.SemaphoreType.DMA,
            pltpu.SemaphoreType.DMA,
            pltpu.SemaphoreType.DMA,
            pltpu.SemaphoreType.DMA,
        ],
    )
    def k(xu, xm, ei, out, iu_all, im_all,
          ru0, ru1, rm0, rm1, o0, o1, sg0, sg1, so0, so1):
        wid = lax.axis_index("s") * NC + lax.axis_index("c")
        n_tail = T - Tf

        def hbase(t):
            return jnp.where(
                t < Tf,
                (wid * Tf + t) * C,
                E - ((T - t) * NW - wid) * C,
            )

        pltpu.sync_copy(ei.at[0, pl.ds(wid * Tf * C, Tf * C)],
                        iu_all.at[pl.ds(0, Tf * C)])
        pltpu.sync_copy(ei.at[1, pl.ds(wid * Tf * C, Tf * C)],
                        im_all.at[pl.ds(0, Tf * C)])
        for j in range(n_tail):
            tb = E - ((n_tail - j) * NW - wid) * C
            pltpu.sync_copy(ei.at[0, pl.ds(tb, C)],
                            iu_all.at[pl.ds((Tf + j) * C, C)])
            pltpu.sync_copy(ei.at[1, pl.ds(tb, C)],
                            im_all.at[pl.ds((Tf + j) * C, C)])

        bufs = ((ru0, rm0, o0, sg0, so0), (ru1, rm1, o1, sg1, so1))

        def gathers(t, b):
            ruv, rmv, _, sg, _ = bufs[b]
            cu = pltpu.make_async_copy(xu.at[iu_all.at[pl.ds(t * C, C)]], ruv, sg)
            cm = pltpu.make_async_copy(xm.at[im_all.at[pl.ds(t * C, C)]], rmv, sg)
            return cu, cm

        for b in range(2):
            cu, cm = gathers(b, b)
            cu.start()
            cm.start()

        lanes = lax.iota(jnp.int32, L)
        iE = (lanes % (L // 2)) * 2
        iO = iE + 1
        lo_mask = lanes < (L // 2)

        def _perm(a, idx_):
            return jnp.take_along_axis(a, idx_, axis=0)

        def _hadd(a, b):
            ta = _perm(a, iE) + _perm(a, iO)
            tb = _perm(b, iE) + _perm(b, iO)
            return jnp.where(lo_mask, ta, tb)

        def outer(i, carry):
            t0 = i * 2
            for b in range(2):
                ruv, rmv, ov, sg, so = bufs[b]
                t = t0 + b
                cu, cm = gathers(t, b)
                cu.wait()
                cm.wait()

                @pl.when(t >= 2)
                def _():
                    pltpu.make_async_copy(
                        ov, out.at[pl.ds(hbase(t - 2), C)], so
                    ).wait()

                def group_body(g, carry2):
                    eb = g * L
                    p = []
                    for j in range(L):
                        e = eb + j
                        v = ruv[e, pl.ds(0, L)] * rmv[e, pl.ds(0, L)]
                        for q in range(1, D // L):
                            v = v + ruv[e, pl.ds(q * L, L)] * rmv[e, pl.ds(q * L, L)]
                        p.append(v)
                    while len(p) > 1:
                        p = [_hadd(p[i2], p[i2 + 1]) for i2 in range(0, len(p), 2)]
                    ov[pl.ds(eb, L)] = p[0]
                    return carry2

                lax.fori_loop(0, C // L, group_body, 0)
                pltpu.make_async_copy(ov, out.at[pl.ds(hbase(t), C)], so).start()

                @pl.when(t + 2 < T)
                def _():
                    cu2, cm2 = gathers(t + 2, b)
                    cu2.start()
                    cm2.start()

            return carry

        lax.fori_loop(0, T // 2, outer, 0)

        for b in range(2):
            _, _, ov, _, so = bufs[b]
            t = T - 2 + b
            pltpu.make_async_copy(ov, out.at[pl.ds(hbase(t), C)], so).wait()

    return k(x_user, x_movie, eli)


def kernel(x_user, x_movie, edge_label_index):
    return _run(x_user, x_movie, edge_label_index)
